# baseline (speedup 1.0000x reference)
import jax
import jax.numpy as jnp
from jax import lax
from jax.experimental import pallas as pl
from jax.experimental.pallas import tpu as pltpu

AU_NUM = 12
_BALANCE = (0.2, 0.3, 0.2, 0.2, 0.5, 0.2, 0.5, 0.2, 0.1, 0.5, 0.2, 0.3)
_B = 512


def _body(x_ref, lab_ref, out_ref, cnt_pos, cnt_neg, run_neg, par, acc):
    phase = pl.program_id(0)
    b = pl.program_id(1)
    nb = pl.num_programs(1)
    col = lax.broadcasted_iota(jnp.int32, (1, AU_NUM), 1)
    bal = jnp.full((1, AU_NUM), 0.2, jnp.float32)
    for i, v in enumerate(_BALANCE):
        if v != 0.2:
            bal = jnp.where(col == i, jnp.float32(v), bal)

    @pl.when((phase == 0) & (b == 0))
    def _init():
        cnt_pos[...] = jnp.zeros((1, AU_NUM), jnp.float32)
        cnt_neg[...] = jnp.zeros((1, AU_NUM), jnp.float32)
        run_neg[...] = jnp.zeros((1, AU_NUM), jnp.float32)
        par[...] = jnp.zeros((1, AU_NUM), jnp.float32)
        acc[...] = jnp.zeros((1, 1), jnp.float32)

    @pl.when(phase == 0)
    def _count():
        lab = lab_ref[...]
        cnt_pos[...] += jnp.sum((lab == 1).astype(jnp.float32), axis=0,
                                keepdims=True)
        cnt_neg[...] += jnp.sum((lab == -1).astype(jnp.float32), axis=0,
                                keepdims=True)

    @pl.when(phase == 1)
    def _loss():
        pos_num = cnt_pos[...]
        neg_num = cnt_neg[...]
        half = (pos_num + neg_num) * bal
        sample = neg_num - jnp.ceil(half)
        branch = (pos_num < half) & (sample >= 1.0)
        safe_pos = jnp.where(pos_num != 0.0, pos_num, 1.0)
        ratio = jnp.minimum(half / safe_pos, 1.0)
        wpos = jnp.where(branch & (pos_num != 0.0), ratio, 1.0)

        x = x_ref[...]
        lab = lab_ref[...]
        is_pos = lab == 1
        is_neg = lab == -1
        pe = (jnp.maximum(x, 0.0) - jnp.where(is_pos, x, 0.0)
              + jnp.log1p(jnp.exp(-jnp.abs(x))))
        s_pos = jnp.sum(jnp.where(is_pos, pe, 0.0), axis=0, keepdims=True)
        s_neg = jnp.sum(jnp.where(is_neg, pe, 0.0), axis=0, keepdims=True)
        negf = is_neg.astype(jnp.float32)
        cnt_b = jnp.sum(negf, axis=0, keepdims=True)
        lo = sample - run_neg[...]

        need = jnp.any(branch & (lo > 0.0) & (lo < cnt_b))

        @pl.when(need)
        def _partial():
            rows = lax.broadcasted_iota(jnp.int32, (_B, _B), 0)
            cols = lax.broadcasted_iota(jnp.int32, (_B, _B), 1)
            tril = (rows >= cols).astype(jnp.float32)
            ranks = jnp.dot(tril, negf,
                            preferred_element_type=jnp.float32)
            par[...] = jnp.sum(
                jnp.where(is_neg & (ranks <= lo), pe, 0.0),
                axis=0, keepdims=True)

        chosen = jnp.where(
            branch,
            jnp.where(lo >= cnt_b, s_neg,
                      jnp.where(lo <= 0.0, 0.0, par[...])),
            0.0)
        acc[...] += jnp.sum(wpos * s_pos + (s_neg - chosen),
                            keepdims=True).reshape(1, 1)
        run_neg[...] += cnt_b

        @pl.when(b == nb - 1)
        def _fin():
            out_ref[...] = acc[...]


def kernel(x, labels):
    n = x.shape[0]
    nb = n // _B
    out = pl.pallas_call(
        _body,
        grid=(2, nb),
        in_specs=[
            pl.BlockSpec((_B, AU_NUM), lambda p, b: (b * p, 0)),
            pl.BlockSpec((_B, AU_NUM), lambda p, b: (b, 0)),
        ],
        out_specs=pl.BlockSpec((1, 1), lambda p, b: (0, 0)),
        out_shape=jax.ShapeDtypeStruct((1, 1), jnp.float32),
        scratch_shapes=[
            pltpu.VMEM((1, AU_NUM), jnp.float32),
            pltpu.VMEM((1, AU_NUM), jnp.float32),
            pltpu.VMEM((1, AU_NUM), jnp.float32),
            pltpu.VMEM((1, AU_NUM), jnp.float32),
            pltpu.VMEM((1, 1), jnp.float32),
        ],
        compiler_params=pltpu.CompilerParams(
            dimension_semantics=("arbitrary", "arbitrary")),
    )(x, labels)
    return out[0, 0]


# speedup vs baseline: 2.7172x; 2.7172x over previous
import jax
import jax.numpy as jnp
from jax import lax
from jax.experimental import pallas as pl
from jax.experimental.pallas import tpu as pltpu

AU_NUM = 12
_BALANCE = (0.2, 0.3, 0.2, 0.2, 0.5, 0.2, 0.5, 0.2, 0.1, 0.5, 0.2, 0.3)
_W = 384
_RB = 256


def _fold_stride12(v):
    for s in (12, 24, 48, 96, 192):
        v = v + jnp.roll(v, s, axis=1)
    return v


def _body(x_ref, lab_ref, out_ref, cnt_pos, cnt_neg, der, run_neg, par, acc):
    phase = pl.program_id(0)
    b = pl.program_id(1)
    nb = pl.num_programs(1)

    @pl.when((phase == 0) & (b == 0))
    def _init():
        cnt_pos[...] = jnp.zeros((1, _W), jnp.float32)
        cnt_neg[...] = jnp.zeros((1, _W), jnp.float32)
        run_neg[...] = jnp.zeros((1, _W), jnp.float32)
        par[...] = jnp.zeros((1, _W), jnp.float32)
        acc[...] = jnp.zeros((1, 1), jnp.float32)

    @pl.when(phase == 0)
    def _count():
        lab = lab_ref[...]
        cnt_pos[...] += jnp.sum((lab == 1).astype(jnp.float32), axis=0,
                                keepdims=True)
        cnt_neg[...] += jnp.sum((lab == -1).astype(jnp.float32), axis=0,
                                keepdims=True)

    @pl.when((phase == 1) & (b == 0))
    def _derive():
        col = lax.broadcasted_iota(jnp.int32, (1, _W), 1) % AU_NUM
        bal = jnp.full((1, _W), 0.2, jnp.float32)
        for i, v in enumerate(_BALANCE):
            if v != 0.2:
                bal = jnp.where(col == i, jnp.float32(v), bal)
        pos_num = _fold_stride12(cnt_pos[...])
        neg_num = _fold_stride12(cnt_neg[...])
        half = (pos_num + neg_num) * bal
        sample = neg_num - jnp.ceil(half)
        branch = (pos_num < half) & (sample >= 1.0)
        safe_pos = jnp.where(pos_num != 0.0, pos_num, 1.0)
        ratio = jnp.minimum(half / safe_pos, 1.0)
        wpos = jnp.where(branch & (pos_num != 0.0), ratio, 1.0)
        der[0:1, :] = wpos
        der[1:2, :] = sample
        der[2:3, :] = branch.astype(jnp.float32)

    @pl.when(phase == 1)
    def _loss():
        wpos = der[0:1, :]
        sample = der[1:2, :]
        branch = der[2:3, :] != 0.0

        x = x_ref[...]
        lab = lab_ref[...]
        is_pos = lab == 1
        is_neg = lab == -1
        pe = (jnp.maximum(x, 0.0) - jnp.where(is_pos, x, 0.0)
              + jnp.log1p(jnp.exp(-jnp.abs(x))))
        s_pos = jnp.sum(jnp.where(is_pos, pe, 0.0), axis=0, keepdims=True)
        s_neg = jnp.sum(jnp.where(is_neg, pe, 0.0), axis=0, keepdims=True)
        negf = is_neg.astype(jnp.float32)
        cnt_b = _fold_stride12(jnp.sum(negf, axis=0, keepdims=True))
        lo = sample - run_neg[...]

        need = jnp.any(branch & (lo > 0.0) & (lo < cnt_b))

        @pl.when(need)
        def _partial():
            l_iota = lax.broadcasted_iota(jnp.int32, (_RB, _W), 1)
            p = negf
            for s in (12, 24, 48, 96, 192):
                p = p + jnp.where(l_iota >= s, jnp.roll(p, s, axis=1), 0.0)
            rt = jnp.where(l_iota >= _W - AU_NUM, p, 0.0)
            for s in (12, 24, 48, 96, 192):
                rt = rt + jnp.where(l_iota < _W - s,
                                    jnp.roll(rt, -s, axis=1), 0.0)
            rows = lax.broadcasted_iota(jnp.int32, (_RB, _RB), 0)
            cols = lax.broadcasted_iota(jnp.int32, (_RB, _RB), 1)
            tril = (rows > cols).astype(jnp.float32)
            rank = jnp.dot(tril, rt, preferred_element_type=jnp.float32) + p
            par[...] = jnp.sum(
                jnp.where(is_neg & (rank <= lo), pe, 0.0),
                axis=0, keepdims=True)

        chosen = jnp.where(
            branch,
            jnp.where(lo >= cnt_b, s_neg,
                      jnp.where(lo <= 0.0, 0.0, par[...])),
            0.0)
        acc[...] += jnp.sum(wpos * s_pos + (s_neg - chosen),
                            keepdims=True).reshape(1, 1)
        run_neg[...] += cnt_b

        @pl.when(b == nb - 1)
        def _fin():
            out_ref[...] = acc[...]


def kernel(x, labels):
    n = x.shape[0]
    rows = n * AU_NUM // _W
    nb = rows // _RB
    xf = x.reshape(rows, _W)
    labf = labels.reshape(rows, _W)
    out = pl.pallas_call(
        _body,
        grid=(2, nb),
        in_specs=[
            pl.BlockSpec((_RB, _W), lambda p, b: (b * p, 0)),
            pl.BlockSpec((_RB, _W), lambda p, b: (b, 0)),
        ],
        out_specs=pl.BlockSpec((1, 1), lambda p, b: (0, 0)),
        out_shape=jax.ShapeDtypeStruct((1, 1), jnp.float32),
        scratch_shapes=[
            pltpu.VMEM((1, _W), jnp.float32),
            pltpu.VMEM((1, _W), jnp.float32),
            pltpu.VMEM((3, _W), jnp.float32),
            pltpu.VMEM((1, _W), jnp.float32),
            pltpu.VMEM((1, _W), jnp.float32),
            pltpu.VMEM((1, 1), jnp.float32),
        ],
        compiler_params=pltpu.CompilerParams(
            dimension_semantics=("arbitrary", "arbitrary")),
    )(xf, labf)
    return out[0, 0]
